# baseline (speedup 1.0000x reference)
import functools

import jax
import jax.numpy as jnp
from jax import lax
from jax.experimental import pallas as pl
from jax.experimental.pallas import tpu as pltpu
from jax.experimental.pallas import tpu_sc as plsc

NUM_EXPERTS = 8
DIM = 1024
HIDDEN_DIM = 4096
T = 32
H_BLK = 2048
N_HBLK = HIDDEN_DIM // H_BLK


def _dispatch_sc(idx_hbm, x_hbm, out_hbm, zero_v, x_v, idx_v, dst_v, sem):
    c = lax.axis_index("c")
    s = lax.axis_index("s")
    wid = s * 2 + c

    @pl.when(wid == 0)
    def _():
        def _zero_row(t, _):
            def _zero_chunk(cc, __):
                zero_v[t, pl.ds(cc * 16, 16)] = jnp.zeros((16,), jnp.float32)
                return __
            return lax.fori_loop(0, DIM // 16, _zero_chunk, _)
        lax.fori_loop(0, T, _zero_row, 0)

        pltpu.sync_copy(x_hbm, x_v)
        pltpu.sync_copy(idx_hbm, idx_v)

        iota16 = lax.iota(jnp.int32, 16)
        a = idx_v[pl.ds(0, 16)]
        dst_v[pl.ds(0, 16)] = a * T + iota16
        b = idx_v[pl.ds(16, 16)]
        dst_v[pl.ds(16, 16)] = b * T + (iota16 + 16)

        for e in range(NUM_EXPERTS):
            pltpu.sync_copy(zero_v, out_hbm.at[pl.ds(e * T, T)])

        pltpu.async_copy(x_v, out_hbm.at[dst_v], sem).wait()


def _moe_kernel(xd_ref, fc1_ref, fc2_ref, out_ref):
    e = pl.program_id(0)
    hb = pl.program_id(1)

    @pl.when(jnp.logical_and(e == 0, hb == 0))
    def _init():
        out_ref[...] = jnp.zeros_like(out_ref)

    h = jax.lax.dot_general(
        xd_ref[0], fc1_ref[0],
        dimension_numbers=(((1,), (1,)), ((), ())),
        preferred_element_type=jnp.float32,
    )
    h = h * jax.nn.sigmoid(h)
    out_ref[...] += jax.lax.dot_general(
        h, fc2_ref[0],
        dimension_numbers=(((1,), (1,)), ((), ())),
        preferred_element_type=jnp.float32,
    )


@jax.jit
def kernel(x, expert_idx, fc1_weight, fc2_weight):
    idx = expert_idx.astype(jnp.int32)
    mesh = plsc.VectorSubcoreMesh(core_axis_name="c", subcore_axis_name="s")
    dispatch = functools.partial(
        pl.kernel,
        mesh=mesh,
        out_type=jax.ShapeDtypeStruct((NUM_EXPERTS * T, DIM), jnp.float32),
        scratch_types=[
            pltpu.VMEM((T, DIM), jnp.float32),
            pltpu.VMEM((T, DIM), jnp.float32),
            pltpu.VMEM((T,), jnp.int32),
            pltpu.VMEM((T,), jnp.int32),
            pltpu.SemaphoreType.DMA,
        ],
    )(_dispatch_sc)
    xd = dispatch(idx, x).reshape(NUM_EXPERTS, T, DIM)

    grid = (NUM_EXPERTS, N_HBLK)
    return pl.pallas_call(
        _moe_kernel,
        grid=grid,
        in_specs=[
            pl.BlockSpec((1, T, DIM), lambda e, hb: (e, 0, 0)),
            pl.BlockSpec((1, H_BLK, DIM), lambda e, hb: (e, hb, 0)),
            pl.BlockSpec((1, DIM, H_BLK), lambda e, hb: (e, 0, hb)),
        ],
        out_specs=pl.BlockSpec((T, DIM), lambda e, hb: (0, 0)),
        out_shape=jax.ShapeDtypeStruct((T, DIM), jnp.float32),
        compiler_params=pltpu.CompilerParams(
            dimension_semantics=("arbitrary", "arbitrary"),
        ),
    )(xd, fc1_weight, fc2_weight)


# speedup vs baseline: 1.4634x; 1.4634x over previous
import functools

import jax
import jax.numpy as jnp
from jax.experimental import pallas as pl
from jax.experimental.pallas import tpu as pltpu

NUM_EXPERTS = 8
DIM = 1024
HIDDEN_DIM = 4096
T = 32
H_BLK = 2048
N_HBLK = HIDDEN_DIM // H_BLK
N_STEPS = NUM_EXPERTS * N_HBLK


def _moe_kernel(idx_ref, x_ref, fc1_ref, fc2_ref, out_ref, h_ref):
    s = pl.program_id(0)

    @pl.when(s == 0)
    def _init():
        out_ref[...] = jnp.zeros_like(out_ref)

    @pl.when(s > 0)
    def _fc2_prev():
        out_ref[...] += jax.lax.dot_general(
            h_ref[(s - 1) % 2], fc2_ref[0],
            dimension_numbers=(((1,), (1,)), ((), ())),
            preferred_element_type=jnp.float32,
        )

    @pl.when(s < N_STEPS)
    def _fc1_cur():
        e = s // N_HBLK
        mask = idx_ref[...] == e
        xm = jnp.where(mask, x_ref[...], 0.0)
        h = jax.lax.dot_general(
            xm, fc1_ref[0],
            dimension_numbers=(((1,), (1,)), ((), ())),
            preferred_element_type=jnp.float32,
        )
        h_ref[s % 2] = h * jax.nn.sigmoid(h)


@jax.jit
def kernel(x, expert_idx, fc1_weight, fc2_weight):
    idx2d = expert_idx.astype(jnp.int32).reshape(T, 1)

    def fc1_map(s):
        sc = jnp.minimum(s, N_STEPS - 1)
        return (sc // N_HBLK, sc % N_HBLK, 0)

    def fc2_map(s):
        sp = jnp.maximum(s - 1, 0)
        return (sp // N_HBLK, 0, sp % N_HBLK)

    return pl.pallas_call(
        _moe_kernel,
        grid=(N_STEPS + 1,),
        in_specs=[
            pl.BlockSpec((T, 1), lambda s: (0, 0)),
            pl.BlockSpec((T, DIM), lambda s: (0, 0)),
            pl.BlockSpec((1, H_BLK, DIM), fc1_map),
            pl.BlockSpec((1, DIM, H_BLK), fc2_map),
        ],
        out_specs=pl.BlockSpec((T, DIM), lambda s: (0, 0)),
        out_shape=jax.ShapeDtypeStruct((T, DIM), jnp.float32),
        scratch_shapes=[pltpu.VMEM((2, T, H_BLK), jnp.float32)],
        compiler_params=pltpu.CompilerParams(
            dimension_semantics=("arbitrary",),
        ),
    )(idx2d, x, fc1_weight, fc2_weight)


# speedup vs baseline: 1.4712x; 1.0053x over previous
import functools

import jax
import jax.numpy as jnp
from jax.experimental import pallas as pl
from jax.experimental.pallas import tpu as pltpu

NUM_EXPERTS = 8
DIM = 1024
HIDDEN_DIM = 4096
T = 32
H_BLK = 2048
N_HBLK = HIDDEN_DIM // H_BLK


def _moe_kernel(idx_ref, x_ref, fc1_ref, fc2_ref, out_ref):
    e = pl.program_id(0)
    hb = pl.program_id(1)

    @pl.when(jnp.logical_and(e == 0, hb == 0))
    def _init():
        out_ref[...] = jnp.zeros_like(out_ref)

    mask = idx_ref[...] == e
    xm = jnp.where(mask, x_ref[...], 0.0)
    h = jax.lax.dot_general(
        xm, fc1_ref[0],
        dimension_numbers=(((1,), (1,)), ((), ())),
        preferred_element_type=jnp.float32,
    )
    h = h * jax.nn.sigmoid(h)
    out_ref[...] += jax.lax.dot_general(
        h, fc2_ref[0],
        dimension_numbers=(((1,), (1,)), ((), ())),
        preferred_element_type=jnp.float32,
    )


@jax.jit
def kernel(x, expert_idx, fc1_weight, fc2_weight):
    idx2d = expert_idx.astype(jnp.int32).reshape(T, 1)
    grid = (NUM_EXPERTS, N_HBLK)
    return pl.pallas_call(
        _moe_kernel,
        grid=grid,
        in_specs=[
            pl.BlockSpec((T, 1), lambda e, hb: (0, 0)),
            pl.BlockSpec((T, DIM), lambda e, hb: (0, 0)),
            pl.BlockSpec((1, H_BLK, DIM), lambda e, hb: (e, hb, 0)),
            pl.BlockSpec((1, DIM, H_BLK), lambda e, hb: (e, 0, hb)),
        ],
        out_specs=pl.BlockSpec((T, DIM), lambda e, hb: (0, 0)),
        out_shape=jax.ShapeDtypeStruct((T, DIM), jnp.float32),
        compiler_params=pltpu.CompilerParams(
            dimension_semantics=("arbitrary", "arbitrary"),
        ),
    )(idx2d, x, fc1_weight, fc2_weight)
